# baseline (speedup 1.0000x reference)
import jax
import jax.numpy as jnp
from jax.experimental import pallas as pl
from jax.experimental.pallas import tpu as pltpu

_PROPOSAL_COUNT = 2000
_PRE_NMS = 6000
_PAD = 6144
_ROWS = 48
_OROWS = 16
_NMS_THR = 0.7


def _nms_kernel(ag, dg, sc, oy1, ox1, oy2, ox2, y1s, x1s, y2s, x2s, ars):
    y1a = ag[0, 0]
    x1a = ag[0, 1]
    y2a = ag[0, 2]
    x2a = ag[0, 3]
    dy = dg[0, 0] * 0.1
    dx = dg[0, 1] * 0.1
    dh = dg[0, 2] * 0.2
    dw = dg[0, 3] * 0.2
    h = y2a - y1a
    w = x2a - x1a
    cy = y1a + 0.5 * h + dy * h
    cx = x1a + 0.5 * w + dx * w
    h = h * jnp.exp(dh)
    w = w * jnp.exp(dw)
    y1u = cy - 0.5 * h
    x1u = cx - 0.5 * w
    y1 = jnp.clip(y1u, 0.0, 1.0)
    x1 = jnp.clip(x1u, 0.0, 1.0)
    y2 = jnp.clip(y1u + h, 0.0, 1.0)
    x2 = jnp.clip(x1u + w, 0.0, 1.0)
    areas = (y2 - y1) * (x2 - x1)
    y1s[...] = y1
    x1s[...] = x1
    y2s[...] = y2
    x2s[...] = x2
    ars[...] = areas

    r = jax.lax.broadcasted_iota(jnp.int32, (_ROWS, 128), 0)
    c = jax.lax.broadcasted_iota(jnp.int32, (_ROWS, 128), 1)
    idxg = r * 128 + c
    lane = jax.lax.broadcasted_iota(jnp.int32, (1, 128), 1)

    zeros_out = jnp.zeros((_OROWS, 128), jnp.float32)
    oy1[0] = zeros_out
    ox1[0] = zeros_out
    oy2[0] = zeros_out
    ox2[0] = zeros_out

    neg_inf = jnp.float32(-jnp.inf)
    big = jnp.int32(0x7FFFFFFF)

    def pick(plane_ref, rowi, cmask):
        row = plane_ref[pl.ds(rowi, 1), :]
        return jnp.sum(jnp.where(cmask, row, 0.0))

    def body(k, sv):
        idx = jnp.min(jnp.where(sv > neg_inf, idxg, big))
        ok = idx < big
        rowi = jnp.where(ok, idx, 0) // 128
        coli = idx % 128
        cmask = lane == coli
        by1 = pick(y1s, rowi, cmask)
        bx1 = pick(x1s, rowi, cmask)
        by2 = pick(y2s, rowi, cmask)
        bx2 = pick(x2s, rowi, cmask)
        barea = pick(ars, rowi, cmask)
        yy1 = jnp.maximum(y1, by1)
        xx1 = jnp.maximum(x1, bx1)
        yy2 = jnp.minimum(y2, by2)
        xx2 = jnp.minimum(x2, bx2)
        inter = jnp.maximum(yy2 - yy1, 0.0) * jnp.maximum(xx2 - xx1, 0.0)
        iou = inter / (areas + barea - inter + 1e-8)
        supp = (iou > _NMS_THR) | (idxg == idx)
        sv = jnp.where(supp, neg_inf, sv)
        orow = k // 128
        omask = (lane == (k % 128)) & ok
        oy1[0, pl.ds(orow, 1), :] = jnp.where(omask, by1, oy1[0, pl.ds(orow, 1), :])
        ox1[0, pl.ds(orow, 1), :] = jnp.where(omask, bx1, ox1[0, pl.ds(orow, 1), :])
        oy2[0, pl.ds(orow, 1), :] = jnp.where(omask, by2, oy2[0, pl.ds(orow, 1), :])
        ox2[0, pl.ds(orow, 1), :] = jnp.where(omask, bx2, ox2[0, pl.ds(orow, 1), :])
        return sv

    jax.lax.fori_loop(0, _PROPOSAL_COUNT, body, sc[0])


def kernel(rpn_probs, rpn_bbox, anchors):
    b = rpn_probs.shape[0]
    scores = rpn_probs[:, :, 1]
    top_scores, ix = jax.lax.top_k(scores, _PRE_NMS)
    deltas_g = jnp.take_along_axis(rpn_bbox, ix[:, :, None], axis=1)
    anchors_g = jnp.take_along_axis(anchors, ix[:, :, None], axis=1)

    pad = _PAD - _PRE_NMS
    sc = jnp.pad(top_scores, ((0, 0), (0, pad)), constant_values=-jnp.inf)
    ag = jnp.pad(anchors_g, ((0, 0), (0, pad), (0, 0)))
    dg = jnp.pad(deltas_g, ((0, 0), (0, pad), (0, 0)))
    ag = ag.transpose(0, 2, 1).reshape(b, 4, _ROWS, 128)
    dg = dg.transpose(0, 2, 1).reshape(b, 4, _ROWS, 128)
    sc = sc.reshape(b, _ROWS, 128)

    out_sds = jax.ShapeDtypeStruct((b, _OROWS, 128), jnp.float32)
    outs = pl.pallas_call(
        _nms_kernel,
        grid=(b,),
        in_specs=[
            pl.BlockSpec((1, 4, _ROWS, 128), lambda i: (i, 0, 0, 0)),
            pl.BlockSpec((1, 4, _ROWS, 128), lambda i: (i, 0, 0, 0)),
            pl.BlockSpec((1, _ROWS, 128), lambda i: (i, 0, 0)),
        ],
        out_specs=[
            pl.BlockSpec((1, _OROWS, 128), lambda i: (i, 0, 0)),
            pl.BlockSpec((1, _OROWS, 128), lambda i: (i, 0, 0)),
            pl.BlockSpec((1, _OROWS, 128), lambda i: (i, 0, 0)),
            pl.BlockSpec((1, _OROWS, 128), lambda i: (i, 0, 0)),
        ],
        out_shape=[out_sds, out_sds, out_sds, out_sds],
        scratch_shapes=[pltpu.VMEM((_ROWS, 128), jnp.float32)] * 5,
    )(ag, dg, sc)

    py1, px1, py2, px2 = [
        o.reshape(b, _OROWS * 128)[:, :_PROPOSAL_COUNT] for o in outs
    ]
    return jnp.stack([py1, px1, py2, px2], axis=-1)


# speedup vs baseline: 1.2516x; 1.2516x over previous
import jax
import jax.numpy as jnp
from jax.experimental import pallas as pl
from jax.experimental.pallas import tpu as pltpu

_PROPOSAL_COUNT = 2000
_PRE_NMS = 6000
_PAD = 6144
_ROWS = 48
_OROWS = 16
_NMS_THR = 0.7
_BATCH = 2


def _nms_kernel(ag, dg, sc, oy1, ox1, oy2, ox2, y1s, x1s, y2s, x2s, ars):
    y1a = ag[0]
    x1a = ag[1]
    y2a = ag[2]
    x2a = ag[3]
    dy = dg[0] * 0.1
    dx = dg[1] * 0.1
    dh = dg[2] * 0.2
    dw = dg[3] * 0.2
    h = y2a - y1a
    w = x2a - x1a
    cy = y1a + 0.5 * h + dy * h
    cx = x1a + 0.5 * w + dx * w
    h = h * jnp.exp(dh)
    w = w * jnp.exp(dw)
    y1u = cy - 0.5 * h
    x1u = cx - 0.5 * w
    y1 = jnp.clip(y1u, 0.0, 1.0)
    x1 = jnp.clip(x1u, 0.0, 1.0)
    y2 = jnp.clip(y1u + h, 0.0, 1.0)
    x2 = jnp.clip(x1u + w, 0.0, 1.0)
    areas = (y2 - y1) * (x2 - x1)
    y1s[...] = y1
    x1s[...] = x1
    y2s[...] = y2
    x2s[...] = x2
    ars[...] = areas

    rows = _BATCH * _ROWS
    r = jax.lax.broadcasted_iota(jnp.int32, (rows, 128), 0)
    c = jax.lax.broadcasted_iota(jnp.int32, (rows, 128), 1)
    idxg = r * 128 + c
    lane = jax.lax.broadcasted_iota(jnp.int32, (1, 128), 1)
    img0 = r < _ROWS

    zeros_out = jnp.zeros((_BATCH * _OROWS, 128), jnp.float32)
    oy1[...] = zeros_out
    ox1[...] = zeros_out
    oy2[...] = zeros_out
    ox2[...] = zeros_out

    neg_inf = jnp.float32(-jnp.inf)
    big = jnp.int32(0x7FFFFFFF)

    def pick(plane_ref, rowi, cmask):
        row = plane_ref[pl.ds(rowi, 1), :]
        return jnp.sum(jnp.where(cmask, row, 0.0))

    def body(k, sv):
        valid = sv > neg_inf
        masked = jnp.where(valid, idxg, big)
        idx0 = jnp.min(jnp.where(img0, masked, big))
        idx1 = jnp.min(jnp.where(img0, big, masked))
        ok0 = idx0 < big
        ok1 = idx1 < big
        row0 = jnp.where(ok0, idx0, 0) // 128
        row1 = jnp.where(ok1, idx1, _ROWS * 128) // 128
        cm0 = lane == (idx0 % 128)
        cm1 = lane == (idx1 % 128)
        by1 = jnp.where(img0, pick(y1s, row0, cm0), pick(y1s, row1, cm1))
        bx1 = jnp.where(img0, pick(x1s, row0, cm0), pick(x1s, row1, cm1))
        by2 = jnp.where(img0, pick(y2s, row0, cm0), pick(y2s, row1, cm1))
        bx2 = jnp.where(img0, pick(x2s, row0, cm0), pick(x2s, row1, cm1))
        bar = jnp.where(img0, pick(ars, row0, cm0), pick(ars, row1, cm1))
        yy1 = jnp.maximum(y1, by1)
        xx1 = jnp.maximum(x1, bx1)
        yy2 = jnp.minimum(y2, by2)
        xx2 = jnp.minimum(x2, bx2)
        inter = jnp.maximum(yy2 - yy1, 0.0) * jnp.maximum(xx2 - xx1, 0.0)
        iou = inter / (areas + bar - inter + 1e-8)
        supp = (iou > _NMS_THR) | (idxg == idx0) | (idxg == idx1)
        sv = jnp.where(supp, neg_inf, sv)
        orow = k // 128
        om0 = (lane == (k % 128)) & ok0
        om1 = (lane == (k % 128)) & ok1
        b1 = pl.ds(orow, 1)
        b2 = pl.ds(_OROWS + orow, 1)
        for oref, p0, p1 in (
            (oy1, pick(y1s, row0, cm0), pick(y1s, row1, cm1)),
            (ox1, pick(x1s, row0, cm0), pick(x1s, row1, cm1)),
            (oy2, pick(y2s, row0, cm0), pick(y2s, row1, cm1)),
            (ox2, pick(x2s, row0, cm0), pick(x2s, row1, cm1)),
        ):
            oref[b1, :] = jnp.where(om0, p0, oref[b1, :])
            oref[b2, :] = jnp.where(om1, p1, oref[b2, :])
        return sv

    jax.lax.fori_loop(0, _PROPOSAL_COUNT, body, sc[...])


def kernel(rpn_probs, rpn_bbox, anchors):
    b = rpn_probs.shape[0]
    scores = rpn_probs[:, :, 1]
    top_scores, ix = jax.lax.top_k(scores, _PRE_NMS)
    deltas_g = jnp.take_along_axis(rpn_bbox, ix[:, :, None], axis=1)
    anchors_g = jnp.take_along_axis(anchors, ix[:, :, None], axis=1)

    pad = _PAD - _PRE_NMS
    sc = jnp.pad(top_scores, ((0, 0), (0, pad)), constant_values=-jnp.inf)
    ag = jnp.pad(anchors_g, ((0, 0), (0, pad), (0, 0)))
    dg = jnp.pad(deltas_g, ((0, 0), (0, pad), (0, 0)))
    ag = ag.transpose(0, 2, 1).reshape(b, 4, _ROWS, 128)
    dg = dg.transpose(0, 2, 1).reshape(b, 4, _ROWS, 128)
    ag = ag.transpose(1, 0, 2, 3).reshape(4, b * _ROWS, 128)
    dg = dg.transpose(1, 0, 2, 3).reshape(4, b * _ROWS, 128)
    sc = sc.reshape(b * _ROWS, 128)

    out_sds = jax.ShapeDtypeStruct((b * _OROWS, 128), jnp.float32)
    outs = pl.pallas_call(
        _nms_kernel,
        out_shape=[out_sds, out_sds, out_sds, out_sds],
        scratch_shapes=[pltpu.VMEM((b * _ROWS, 128), jnp.float32)] * 5,
    )(ag, dg, sc)

    py1, px1, py2, px2 = [
        o.reshape(b, _OROWS * 128)[:, :_PROPOSAL_COUNT] for o in outs
    ]
    return jnp.stack([py1, px1, py2, px2], axis=-1)
